# baseline (speedup 1.0000x reference)
import jax
import jax.numpy as jnp
from jax.experimental import pallas as pl

K = 1024
D = 256
B = 16
C = 576
N = B * C


def _vq_batch_kernel(z_ref, esq_ref, ehi_ref, elo_ref, emb_ref,
                     zq_ref, idx_ref, dmin_ref):
    zt = z_ref[0]
    zrow = zt.T
    zsq = jnp.sum(zrow * zrow, axis=1)
    mt = jax.lax.dot_general(
        emb_ref[...], zt, (((1,), (0,)), ((), ())),
        preferred_element_type=jnp.float32)
    d = (esq_ref[...][:, None] + zsq[None, :]) - 2.0 * mt
    dmin = jnp.min(d, axis=0)
    iota = jax.lax.broadcasted_iota(jnp.int32, (K, C), 0).astype(jnp.float32)
    sel = jnp.where(d == dmin[None, :], iota, jnp.float32(K))
    idx_f = jnp.min(sel, axis=0)
    oh = (iota == idx_f[None, :]).astype(jnp.float32)
    zq = (jax.lax.dot_general(ehi_ref[...], oh, (((0,), (0,)), ((), ())),
                              preferred_element_type=jnp.float32)
          + jax.lax.dot_general(elo_ref[...], oh, (((0,), (0,)), ((), ())),
                                preferred_element_type=jnp.float32))
    zq_ref[0] = zq
    idx_ref[0, 0] = idx_f.astype(jnp.int32)
    dmin_ref[0, 0] = dmin


def kernel(z, emb):
    Bz, Dd, H, W = z.shape
    z3 = z.reshape(Bz, Dd, H * W)
    esq = jnp.sum(emb ** 2, axis=1)
    e_hi = jax.lax.bitcast_convert_type(
        jax.lax.bitcast_convert_type(emb, jnp.uint32) & jnp.uint32(0xFFFF0000),
        jnp.float32)
    e_lo = emb - e_hi

    zq3, idx, dmin = pl.pallas_call(
        _vq_batch_kernel,
        grid=(Bz,),
        in_specs=[
            pl.BlockSpec((1, Dd, H * W), lambda b: (b, 0, 0)),
            pl.BlockSpec((K,), lambda b: (0,)),
            pl.BlockSpec((K, D), lambda b: (0, 0)),
            pl.BlockSpec((K, D), lambda b: (0, 0)),
            pl.BlockSpec((K, D), lambda b: (0, 0)),
        ],
        out_specs=[
            pl.BlockSpec((1, Dd, H * W), lambda b: (b, 0, 0)),
            pl.BlockSpec((1, 1, H * W), lambda b: (b, 0, 0)),
            pl.BlockSpec((1, 1, H * W), lambda b: (b, 0, 0)),
        ],
        out_shape=[
            jax.ShapeDtypeStruct((Bz, Dd, H * W), jnp.float32),
            jax.ShapeDtypeStruct((Bz, 1, H * W), jnp.int32),
            jax.ShapeDtypeStruct((Bz, 1, H * W), jnp.float32),
        ],
    )(z3, esq, e_hi, e_lo, emb)

    commitment_loss = 0.25 * (jnp.sum(dmin) / (N * D))
    z_q_out = zq3.reshape(Bz, Dd, H, W)
    indices_out = idx.reshape(Bz, H, W)
    return (z_q_out, commitment_loss, indices_out)


# speedup vs baseline: 1.4933x; 1.4933x over previous
import jax
import jax.numpy as jnp
from jax.experimental import pallas as pl

K = 1024
D = 256
N = 9216
BLK = 1024
NB = N // BLK


def _vq_block_kernel(z_ref, esq_ref, ehi_ref, elo_ref, emb_ref,
                     zq_ref, idx_ref, dmin_ref):
    zb = z_ref[...]
    zsq = jnp.sum(zb * zb, axis=1)
    m = jax.lax.dot_general(
        zb, emb_ref[...], (((1,), (1,)), ((), ())),
        preferred_element_type=jnp.float32)
    d = (zsq[:, None] + esq_ref[...][None, :]) - 2.0 * m
    dmin = jnp.min(d, axis=1)
    iota = jax.lax.broadcasted_iota(jnp.int32, (BLK, K), 1).astype(jnp.float32)
    sel = jnp.where(d == dmin[:, None], iota, jnp.float32(K))
    idx_f = jnp.min(sel, axis=1)
    oh = (iota == idx_f[:, None]).astype(jnp.float32)
    zq = (jax.lax.dot_general(oh, ehi_ref[...], (((1,), (0,)), ((), ())),
                              preferred_element_type=jnp.float32)
          + jax.lax.dot_general(oh, elo_ref[...], (((1,), (0,)), ((), ())),
                                preferred_element_type=jnp.float32))
    zq_ref[...] = zq
    idx_ref[...] = idx_f.astype(jnp.int32)
    dmin_ref[...] = dmin


def kernel(z, emb):
    B, Dd, H, W = z.shape
    z_flat = jnp.transpose(z, (0, 2, 3, 1)).reshape(-1, Dd)
    esq = jnp.sum(emb ** 2, axis=1)
    e_hi = jax.lax.bitcast_convert_type(
        jax.lax.bitcast_convert_type(emb, jnp.uint32) & jnp.uint32(0xFFFF0000),
        jnp.float32)
    e_lo = emb - e_hi

    zq_flat, idx, dmin = pl.pallas_call(
        _vq_block_kernel,
        grid=(NB,),
        in_specs=[
            pl.BlockSpec((BLK, D), lambda i: (i, 0)),
            pl.BlockSpec((K,), lambda i: (0,)),
            pl.BlockSpec((K, D), lambda i: (0, 0)),
            pl.BlockSpec((K, D), lambda i: (0, 0)),
            pl.BlockSpec((K, D), lambda i: (0, 0)),
        ],
        out_specs=[
            pl.BlockSpec((BLK, D), lambda i: (i, 0)),
            pl.BlockSpec((BLK,), lambda i: (i,)),
            pl.BlockSpec((BLK,), lambda i: (i,)),
        ],
        out_shape=[
            jax.ShapeDtypeStruct((N, D), jnp.float32),
            jax.ShapeDtypeStruct((N,), jnp.int32),
            jax.ShapeDtypeStruct((N,), jnp.float32),
        ],
    )(z_flat, esq, e_hi, e_lo, emb)

    commitment_loss = 0.25 * (jnp.sum(dmin) / (N * D))
    z_q_out = jnp.transpose(zq_flat.reshape(B, H, W, Dd), (0, 3, 1, 2))
    indices_out = idx.reshape(B, H, W)
    return (z_q_out, commitment_loss, indices_out)
